# baseline (speedup 1.0000x reference)
import jax
import jax.numpy as jnp
from jax.experimental import pallas as pl
from jax.experimental.pallas import tpu as pltpu

_N = 256
_S = 64


def _fused_kernel(locs_ref, shot_ref, WgT_ref, asrc_ref, adst_ref, bg_ref,
                  Wtcn_ref, bt_ref, Ws1T_ref, bs1_ref, Ws2T_ref, bs2_ref,
                  Wl1T_ref, bl1_ref, Wl2T_ref, bl2_ref,
                  Wrl_ref, brl_ref, Wrs_ref, brs_ref,
                  so_ref, lo_ref, rlocs_ref, rshot_ref):
    f32 = jnp.float32
    srow = jax.lax.broadcasted_iota(jnp.int32, (_N, _N), 0)
    dcol = jax.lax.broadcasted_iota(jnp.int32, (_N, _N), 1)
    cidx = jax.lax.broadcasted_iota(jnp.int32, (16, _N), 1)
    ones_row = jnp.ones((1, _N), f32)
    rep = lambda r: jnp.dot(jnp.transpose(r), ones_row,
                            preferred_element_type=f32)
    Asrc = rep(asrc_ref[...])
    Bg = rep(bg_ref[...])
    Bt = rep(bt_ref[...])
    Bs1 = rep(bs1_ref[...])
    Bl1 = rep(bl1_ref[...])
    Rrs = rep(Wrs_ref[...])
    Brs = rep(brs_ref[...])
    Rrl = rep(Wrl_ref[...])
    Brl = rep(brl_ref[...])
    WgT = WgT_ref[...]
    Wt0 = Wtcn_ref[0]
    Wt1 = Wtcn_ref[1]
    Wt2 = Wtcn_ref[2]

    R = range(_S)
    dot = lambda a, b: jnp.dot(a, b, preferred_element_type=f32)

    hs = [dot(WgT, locs_ref[i]) for i in R]
    ads = [dot(adst_ref[...], hs[i]) for i in R]
    ess = [jax.lax.dot_general(hs[i], Asrc, (((0,), (0,)), ((), ())),
                               preferred_element_type=f32) for i in R]
    asr = [dot(asrc_ref[...], hs[i]) for i in R]
    ms = []
    for i in R:
        t = jnp.max(asr[i]) + ads[i]
        ms.append(jnp.where(t >= 0, t, 0.2 * t))
    ps = []
    for i in R:
        t = ess[i] + ads[i]
        t = jnp.where(t >= 0, t, 0.2 * t) - ms[i]
        ps.append(jnp.where(srow == dcol, 0.0, jnp.exp(t)))
    ssums = [jnp.sum(ps[i], axis=0, keepdims=True) for i in R]
    gats = [jnp.maximum(dot(hs[i], ps[i]) / ssums[i] + Bg, 0.0) for i in R]

    s0s = [shot_ref[i] for i in R]
    s1s = [jnp.where(cidx >= 1, pltpu.roll(s, 1, 1), 0.0) for s in s0s]
    s2s = [jnp.where(cidx >= 2, pltpu.roll(s, 2, 1), 0.0) for s in s0s]
    tcns = [jnp.maximum(dot(Wt2, s0s[i]) + dot(Wt1, s1s[i])
                        + dot(Wt0, s2s[i]) + Bt, 0.0) for i in R]

    leaky = lambda z: jnp.where(z >= 0, z, 0.01 * z)
    zss = [leaky(dot(Ws1T_ref[:, 0:16], gats[i])
                 + dot(Ws1T_ref[:, 16:32], tcns[i]) + Bs1) for i in R]
    sos = [dot(Ws2T_ref[...], zss[i]) + bs2_ref[...] for i in R]
    zls = [leaky(dot(Wl1T_ref[:, 0:16], gats[i])
                 + dot(Wl1T_ref[:, 16:32], tcns[i]) + Bl1) for i in R]
    los = [dot(Wl2T_ref[...], zls[i]) + bl2_ref[...] for i in R]

    for i in R:
        so_ref[i] = sos[i]
        lo_ref[i] = los[i]
        rlocs_ref[i] = los[i] * Rrl + Brl
        rshot_ref[i] = sos[i] * Rrs + Brs


def kernel(locs, shot, W_gat, att_src, att_dst, b_gat, W_tcn, b_tcn,
           W_s1, b_s1, W_s2, b_s2, W_l1, b_l1, W_l2, b_l2,
           W_rl, b_rl, W_rs, b_rs):
    B, N, _ = locs.shape
    f32 = jnp.float32

    tr = lambda a: jnp.transpose(a, (0, 2, 1))
    row = lambda v: v.reshape(1, -1)
    args = (
        tr(locs), tr(shot), W_gat.T,
        row(att_src), row(att_dst), row(b_gat),
        jnp.transpose(W_tcn, (2, 0, 1)), row(b_tcn),
        W_s1.T, row(b_s1), W_s2.T, row(b_s2),
        W_l1.T, row(b_l1), W_l2.T, row(b_l2),
        W_rl, row(b_rl), W_rs, row(b_rs),
    )

    batch3 = lambda d: pl.BlockSpec((_S, d, N), lambda b: (b, 0, 0))
    full = lambda a: pl.BlockSpec(a.shape, lambda b: (0,) * a.ndim)
    in_specs = [batch3(2), batch3(16)] + [full(a) for a in args[2:]]

    out_shape = (
        jax.ShapeDtypeStruct((B, 1, N), f32),
        jax.ShapeDtypeStruct((B, 1, N), f32),
        jax.ShapeDtypeStruct((B, 2, N), f32),
        jax.ShapeDtypeStruct((B, 16, N), f32),
    )
    out_specs = (batch3(1), batch3(1), batch3(2), batch3(16))

    outs = pl.pallas_call(
        _fused_kernel,
        grid=(B // _S,),
        in_specs=in_specs,
        out_specs=out_specs,
        out_shape=out_shape,
        compiler_params=pltpu.CompilerParams(
            dimension_semantics=("parallel",),
        ),
    )(*args)
    return tuple(tr(o) for o in outs)


# speedup vs baseline: 1.0290x; 1.0290x over previous
import jax
import jax.numpy as jnp
from jax.experimental import pallas as pl
from jax.experimental.pallas import tpu as pltpu

_N = 256
_S = 64


def _fused_kernel(locs_ref, shot_ref, WgT_ref, asrc_ref, adst_ref, bg_ref,
                  Wtcn_ref, bt_ref, Ws1T_ref, bs1_ref, Ws2T_ref, bs2_ref,
                  Wl1T_ref, bl1_ref, Wl2T_ref, bl2_ref,
                  Wrl_ref, brl_ref, Wrs_ref, brs_ref,
                  so_ref, lo_ref, rlocs_ref, rshot_ref):
    f32 = jnp.float32
    srow = jax.lax.broadcasted_iota(jnp.int32, (_N, _N), 0)
    dcol = jax.lax.broadcasted_iota(jnp.int32, (_N, _N), 1)
    cidx = jax.lax.broadcasted_iota(jnp.int32, (16, _N), 1)
    ones_row = jnp.ones((1, _N), f32)
    rep = lambda r: jnp.dot(jnp.transpose(r), ones_row,
                            preferred_element_type=f32)
    Asrc = rep(asrc_ref[...])
    Bg = rep(bg_ref[...])
    Bt = rep(bt_ref[...])
    Bs1 = rep(bs1_ref[...])
    Bl1 = rep(bl1_ref[...])
    Rrs = rep(Wrs_ref[...])
    Brs = rep(brs_ref[...])
    Rrl = rep(Wrl_ref[...])
    Brl = rep(brl_ref[...])
    WgT = WgT_ref[...]
    Wt0 = Wtcn_ref[0]
    Wt1 = Wtcn_ref[1]
    Wt2 = Wtcn_ref[2]

    R = range(_S)
    dot = lambda a, b: jnp.dot(a, b, preferred_element_type=f32)

    hs = [dot(WgT, locs_ref[i]) for i in R]
    ads = [dot(adst_ref[...], hs[i]) for i in R]
    ess = [jax.lax.dot_general(hs[i], Asrc, (((0,), (0,)), ((), ())),
                               preferred_element_type=f32) for i in R]
    es = [jnp.where(srow == dcol, f32(-1e30),
                    jnp.where(e >= 0, e, 0.2 * e))
          for e in (ess[i] + ads[i] for i in R)]
    ms = [jnp.max(es[i], axis=0, keepdims=True) for i in R]
    ps = [jnp.exp(es[i] - ms[i]) for i in R]
    ssums = [jnp.sum(ps[i], axis=0, keepdims=True) for i in R]
    gats = [jnp.maximum(dot(hs[i], ps[i]) / ssums[i] + Bg, 0.0) for i in R]

    s0s = [shot_ref[i] for i in R]
    s1s = [jnp.where(cidx >= 1, pltpu.roll(s, 1, 1), 0.0) for s in s0s]
    s2s = [jnp.where(cidx >= 2, pltpu.roll(s, 2, 1), 0.0) for s in s0s]
    tcns = [jnp.maximum(dot(Wt2, s0s[i]) + dot(Wt1, s1s[i])
                        + dot(Wt0, s2s[i]) + Bt, 0.0) for i in R]

    leaky = lambda z: jnp.where(z >= 0, z, 0.01 * z)
    zss = [leaky(dot(Ws1T_ref[:, 0:16], gats[i])
                 + dot(Ws1T_ref[:, 16:32], tcns[i]) + Bs1) for i in R]
    sos = [dot(Ws2T_ref[...], zss[i]) + bs2_ref[...] for i in R]
    zls = [leaky(dot(Wl1T_ref[:, 0:16], gats[i])
                 + dot(Wl1T_ref[:, 16:32], tcns[i]) + Bl1) for i in R]
    los = [dot(Wl2T_ref[...], zls[i]) + bl2_ref[...] for i in R]

    for i in R:
        so_ref[i] = sos[i]
        lo_ref[i] = los[i]
        rlocs_ref[i] = los[i] * Rrl + Brl
        rshot_ref[i] = sos[i] * Rrs + Brs


def kernel(locs, shot, W_gat, att_src, att_dst, b_gat, W_tcn, b_tcn,
           W_s1, b_s1, W_s2, b_s2, W_l1, b_l1, W_l2, b_l2,
           W_rl, b_rl, W_rs, b_rs):
    B, N, _ = locs.shape
    f32 = jnp.float32

    tr = lambda a: jnp.transpose(a, (0, 2, 1))
    row = lambda v: v.reshape(1, -1)
    args = (
        tr(locs), tr(shot), W_gat.T,
        row(att_src), row(att_dst), row(b_gat),
        jnp.transpose(W_tcn, (2, 0, 1)), row(b_tcn),
        W_s1.T, row(b_s1), W_s2.T, row(b_s2),
        W_l1.T, row(b_l1), W_l2.T, row(b_l2),
        W_rl, row(b_rl), W_rs, row(b_rs),
    )

    batch3 = lambda d: pl.BlockSpec((_S, d, N), lambda b: (b, 0, 0))
    full = lambda a: pl.BlockSpec(a.shape, lambda b: (0,) * a.ndim)
    in_specs = [batch3(2), batch3(16)] + [full(a) for a in args[2:]]

    out_shape = (
        jax.ShapeDtypeStruct((B, 1, N), f32),
        jax.ShapeDtypeStruct((B, 1, N), f32),
        jax.ShapeDtypeStruct((B, 2, N), f32),
        jax.ShapeDtypeStruct((B, 16, N), f32),
    )
    out_specs = (batch3(1), batch3(1), batch3(2), batch3(16))

    outs = pl.pallas_call(
        _fused_kernel,
        grid=(B // _S,),
        in_specs=in_specs,
        out_specs=out_specs,
        out_shape=out_shape,
        compiler_params=pltpu.CompilerParams(
            dimension_semantics=("parallel",),
        ),
    )(*args)
    return tuple(tr(o) for o in outs)
